# Initial kernel scaffold; baseline (speedup 1.0000x reference)
import jax
import jax.numpy as jnp
from jax.experimental import pallas as pl


def kernel(x, edge_index, W, b):
    raise NotImplementedError("write your pallas kernel here")


# speedup vs baseline: 15.8851x; 15.8851x over previous
import functools

import jax
import jax.numpy as jnp
from jax import lax
from jax.experimental import pallas as pl
from jax.experimental.pallas import tpu as pltpu
from jax.experimental.pallas import tpu_sc as plsc

_N = 10000
_E = 320000
_D = 128
_T = 4
_VTH = 1.0

_NC = 2
_NS = 16
_NW = _NC * _NS
_CHUNK = 128
_NCH = -(-_E // (_NW * _CHUNK))
_EPAD = _NW * _NCH * _CHUNK
_NP = 10240
_RPT = _NP // _NS
_NP8 = _NP * 8
_DPT = _NP8 // _NS


def _sc_mesh():
    return plsc.VectorSubcoreMesh(
        core_axis_name="c", subcore_axis_name="s",
        num_cores=_NC, num_subcores=_NS)


def _deg_body(dst8_hbm, zdeg_hbm, ones_hbm, out_hbm, idx_v, ones_v, deg_sh):
    c = lax.axis_index("c")
    s = lax.axis_index("s")
    w = c * _NS + s
    r0 = s * _DPT
    pltpu.sync_copy(zdeg_hbm.at[pl.ds(r0, _DPT)], deg_sh.at[pl.ds(r0, _DPT)])
    pltpu.sync_copy(ones_hbm, ones_v)
    pltpu.sync_copy(dst8_hbm.at[w], idx_v)
    plsc.subcore_barrier()

    def body(j, carry):
        pltpu.sync_copy(ones_v, deg_sh.at[idx_v.at[j]], add=True)
        return carry

    lax.fori_loop(0, _NCH, body, 0)
    plsc.subcore_barrier()
    pltpu.sync_copy(deg_sh.at[pl.ds(r0, _DPT)], out_hbm.at[c, pl.ds(r0, _DPT)])


def _deg_call(dst8_3d, zdeg, ones):
    k = functools.partial(
        pl.kernel, _deg_body, mesh=_sc_mesh(),
        out_type=jax.ShapeDtypeStruct((_NC, _NP8), jnp.float32),
        scratch_types=[
            pltpu.VMEM((_NCH, _CHUNK), jnp.int32),
            pltpu.VMEM((_CHUNK,), jnp.float32),
            pltpu.VMEM_SHARED((_NP8,), jnp.float32),
        ],
    )()
    return k(dst8_3d, zdeg, ones)


def _msg_body(g_hbm, src_hbm, dst_hbm, zacc_hbm, out_hbm,
              src_v, dst_v, rows_v, acc_sh):
    c = lax.axis_index("c")
    s = lax.axis_index("s")
    w = c * _NS + s
    r0 = s * _RPT

    @pl.when(c == 0)
    def _():
        pltpu.sync_copy(g_hbm.at[pl.ds(r0, _RPT)], acc_sh.at[pl.ds(r0, _RPT)])

    @pl.when(c != 0)
    def _():
        pltpu.sync_copy(zacc_hbm.at[pl.ds(r0, _RPT)],
                        acc_sh.at[pl.ds(r0, _RPT)])

    pltpu.sync_copy(src_hbm.at[w], src_v)
    pltpu.sync_copy(dst_hbm.at[w], dst_v)
    plsc.subcore_barrier()

    def body(j, carry):
        pltpu.sync_copy(g_hbm.at[src_v.at[j]], rows_v)
        pltpu.sync_copy(rows_v, acc_sh.at[dst_v.at[j]], add=True)
        return carry

    lax.fori_loop(0, _NCH, body, 0)
    plsc.subcore_barrier()
    pltpu.sync_copy(acc_sh.at[pl.ds(r0, _RPT)],
                    out_hbm.at[c, pl.ds(r0, _RPT)])


def _msg_call(g, src_3d, dst_3d, zacc):
    k = functools.partial(
        pl.kernel, _msg_body, mesh=_sc_mesh(),
        out_type=jax.ShapeDtypeStruct((_NC, _NP, _D), jnp.float32),
        scratch_types=[
            pltpu.VMEM((_NCH, _CHUNK), jnp.int32),
            pltpu.VMEM((_NCH, _CHUNK), jnp.int32),
            pltpu.VMEM((_CHUNK, _D), jnp.float32),
            pltpu.VMEM_SHARED((_NP, _D), jnp.float32),
        ],
    )()
    return k(g, src_3d, dst_3d, zacc)


def _mm_body(x_ref, deg_ref, w_ref, g_ref):
    d = deg_ref[...]
    deg = d[0] + d[1] + 1.0
    dinv = lax.rsqrt(jnp.maximum(deg[:, 0:1], 1e-12))
    h = jnp.dot(x_ref[...], w_ref[...], preferred_element_type=jnp.float32)
    g_ref[...] = h * dinv


def _mm_call(x_p, deg_t, W):
    RB = 1024
    return pl.pallas_call(
        _mm_body,
        grid=(_NP // RB,),
        in_specs=[
            pl.BlockSpec((RB, _D), lambda i: (i, 0)),
            pl.BlockSpec((_NC, RB, 8), lambda i: (0, i, 0)),
            pl.BlockSpec((_D, _D), lambda i: (0, 0)),
        ],
        out_specs=pl.BlockSpec((RB, _D), lambda i: (i, 0)),
        out_shape=jax.ShapeDtypeStruct((_NP, _D), jnp.float32),
    )(x_p, deg_t, W)


def _fire_body(sp_ref, deg_ref, b_ref, o_ref, z_ref):
    d = deg_ref[...]
    deg = d[0] + d[1] + 1.0
    dinv = lax.rsqrt(jnp.maximum(deg[:, 0:1], 1e-12))
    s = sp_ref[0] + sp_ref[1]
    out = s * dinv + b_ref[...]
    z = jnp.zeros_like(out)
    for t in range(_T):
        z = z + out
        o = (z >= _VTH).astype(jnp.float32)
        z = z * (1.0 - o)
        o_ref[t] = o
        z_ref[t] = z


def _fire_call(sp, deg_t, b2d):
    RB = 1000
    return pl.pallas_call(
        _fire_body,
        grid=(_N // RB,),
        in_specs=[
            pl.BlockSpec((_NC, RB, _D), lambda i: (0, i, 0)),
            pl.BlockSpec((_NC, RB, 8), lambda i: (0, i, 0)),
            pl.BlockSpec((1, _D), lambda i: (0, 0)),
        ],
        out_specs=[
            pl.BlockSpec((_T, RB, _D), lambda i: (0, i, 0)),
            pl.BlockSpec((_T, RB, _D), lambda i: (0, i, 0)),
        ],
        out_shape=[
            jax.ShapeDtypeStruct((_T, _N, _D), jnp.float32),
            jax.ShapeDtypeStruct((_T, _N, _D), jnp.float32),
        ],
    )(sp, deg_t, b2d)


def kernel(x, edge_index, W, b):
    src = edge_index[0]
    dst = edge_index[1]
    npad = _EPAD - _E
    src_p = jnp.concatenate([src, jnp.zeros((npad,), jnp.int32)])
    dst_p = jnp.concatenate([dst, jnp.full((npad,), _NP - 1, jnp.int32)])
    src_3d = src_p.reshape(_NW, _NCH, _CHUNK)
    dst_3d = dst_p.reshape(_NW, _NCH, _CHUNK)
    dst8_3d = dst_3d * 8

    x_p = jnp.pad(x, ((0, _NP - _N), (0, 0)))
    zdeg = jnp.zeros((_NP8,), jnp.float32)
    zacc = jnp.zeros((_NP, _D), jnp.float32)
    ones = jnp.ones((_CHUNK,), jnp.float32)
    b2d = b.reshape(1, _D)

    deg_flat = _deg_call(dst8_3d, zdeg, ones)
    deg_t = deg_flat.reshape(_NC, _NP, 8)
    g = _mm_call(x_p, deg_t, W)
    sp = _msg_call(g, src_3d, dst_3d, zacc)
    o_seq, z_seq = _fire_call(sp, deg_t, b2d)
    return (o_seq, z_seq)
